# baseline (speedup 1.0000x reference)
import functools

import jax
import jax.numpy as jnp
import numpy as np
from jax.experimental import pallas as pl

_PACK_G = 2048


def _pack_matrix(width: int) -> np.ndarray:
    pk = np.zeros((width, width // 16), np.float32)
    j = np.arange(width)
    pk[j, j // 16] = 2.0 ** (j % 16)
    return pk


def _layer1_body(adj_ref, h_ref, hblk_ref, cvec_ref, pk_ref, w1_ref, b1_ref,
                 w2_ref, b2_ref, x_ref, stats_ref, pack_ref):
    adj = adj_ref[...]
    n = adj.shape[1]
    pooled = jnp.dot(adj, h_ref[...], preferred_element_type=jnp.float32)
    pooled = pooled + cvec_ref[...] * hblk_ref[...]
    t = jnp.dot(pooled, w1_ref[...], preferred_element_type=jnp.float32)
    t = jnp.maximum(t + b1_ref[...], 0.0)
    x = jnp.dot(t, w2_ref[...], preferred_element_type=jnp.float32)
    x = x + b2_ref[...]
    x_ref[...] = x
    stats_ref[...] = jnp.stack([jnp.sum(x, axis=0),
                                jnp.sum(x * x, axis=0)])[None]
    parts = []
    for g0 in range(0, n, _PACK_G):
        w = min(_PACK_G, n - g0)
        parts.append(jnp.dot(adj[:, g0:g0 + w], pk_ref[:w, :w // 16],
                             preferred_element_type=jnp.float32))
    pack_ref[...] = jnp.concatenate(parts, axis=1).astype(jnp.int32)


def _layer2_body(pack_ref, hphi_ref, hplo_ref, hblk_ref, cvec_ref, w1_ref,
                 b1_ref, w2_ref, b2_ref, x_ref, stats_ref):
    bits = pack_ref[...]
    bm = bits.shape[0]
    acc = jnp.zeros((bm, hphi_ref.shape[2]), jnp.float32)
    for k in range(16):
        if k:
            bits = jax.lax.shift_right_logical(bits, 1)
        v = (bits & 1).astype(jnp.bfloat16)
        acc = acc + jnp.dot(v, hphi_ref[k], preferred_element_type=jnp.float32)
        acc = acc + jnp.dot(v, hplo_ref[k], preferred_element_type=jnp.float32)
    pooled = acc + cvec_ref[...] * hblk_ref[...]
    t = jnp.dot(pooled, w1_ref[...], preferred_element_type=jnp.float32)
    t = jnp.maximum(t + b1_ref[...], 0.0)
    x = jnp.dot(t, w2_ref[...], preferred_element_type=jnp.float32)
    x = x + b2_ref[...]
    x_ref[...] = x
    stats_ref[...] = jnp.stack([jnp.sum(x, axis=0),
                                jnp.sum(x * x, axis=0)])[None]


def _bn_body(x_ref, stats_ref, gamma_ref, beta_ref, h_ref):
    n = x_ref.shape[0]
    s = jnp.sum(stats_ref[...], axis=0)
    m = s[0:1] * (1.0 / n)
    var = s[1:2] * (1.0 / n) - m * m
    inv = gamma_ref[...] * jax.lax.rsqrt(var + 1e-5)
    h_ref[...] = jnp.maximum((x_ref[...] - m) * inv + beta_ref[...], 0.0)


def _bn_head_body(x_ref, stats_ref, gamma_ref, beta_ref, wp_ref, bp_ref,
                  out_ref):
    n = x_ref.shape[0]
    s = jnp.sum(stats_ref[...], axis=0)
    m = s[0:1] * (1.0 / n)
    var = s[1:2] * (1.0 / n) - m * m
    inv = gamma_ref[...] * jax.lax.rsqrt(var + 1e-5)
    h = jnp.maximum((x_ref[...] - m) * inv + beta_ref[...], 0.0)
    out_ref[...] = jnp.dot(h, wp_ref[...],
                           preferred_element_type=jnp.float32) + bp_ref[...]


def _mlp_specs(d):
    return [
        pl.BlockSpec((1, d), lambda i: (0, 0)),
        pl.BlockSpec((d, d), lambda i: (0, 0)),
        pl.BlockSpec((1, d), lambda i: (0, 0)),
        pl.BlockSpec((d, d), lambda i: (0, 0)),
        pl.BlockSpec((1, d), lambda i: (0, 0)),
    ]


def _layer1(adj, h, cvec, pk, w1, b1, w2, b2, bm):
    n, d = h.shape
    nb = n // bm
    np16 = n // 16
    pkw = pk.shape[0]
    return pl.pallas_call(
        _layer1_body,
        grid=(nb,),
        in_specs=[
            pl.BlockSpec((bm, n), lambda i: (i, 0)),
            pl.BlockSpec((n, d), lambda i: (0, 0)),
            pl.BlockSpec((bm, d), lambda i: (i, 0)),
            pl.BlockSpec((1, d), lambda i: (0, 0)),
            pl.BlockSpec((pkw, pkw // 16), lambda i: (0, 0)),
            pl.BlockSpec((d, d), lambda i: (0, 0)),
            pl.BlockSpec((1, d), lambda i: (0, 0)),
            pl.BlockSpec((d, d), lambda i: (0, 0)),
            pl.BlockSpec((1, d), lambda i: (0, 0)),
        ],
        out_specs=[
            pl.BlockSpec((bm, d), lambda i: (i, 0)),
            pl.BlockSpec((1, 2, d), lambda i: (i, 0, 0)),
            pl.BlockSpec((bm, np16), lambda i: (i, 0)),
        ],
        out_shape=[
            jax.ShapeDtypeStruct((n, d), jnp.float32),
            jax.ShapeDtypeStruct((nb, 2, d), jnp.float32),
            jax.ShapeDtypeStruct((n, np16), jnp.int32),
        ],
    )(adj, h, h, cvec, pk, w1, b1, w2, b2)


def _layer2(pack, hphi, hplo, h, cvec, w1, b1, w2, b2, bm):
    n, d = h.shape
    nb = n // bm
    np16 = n // 16
    return pl.pallas_call(
        _layer2_body,
        grid=(nb,),
        in_specs=[
            pl.BlockSpec((bm, np16), lambda i: (i, 0)),
            pl.BlockSpec((16, np16, d), lambda i: (0, 0, 0)),
            pl.BlockSpec((16, np16, d), lambda i: (0, 0, 0)),
            pl.BlockSpec((bm, d), lambda i: (i, 0)),
        ] + _mlp_specs(d),
        out_specs=[
            pl.BlockSpec((bm, d), lambda i: (i, 0)),
            pl.BlockSpec((1, 2, d), lambda i: (i, 0, 0)),
        ],
        out_shape=[
            jax.ShapeDtypeStruct((n, d), jnp.float32),
            jax.ShapeDtypeStruct((nb, 2, d), jnp.float32),
        ],
    )(pack, hphi, hplo, h, cvec, w1, b1, w2, b2)


def kernel(seq1, adj, W1, b1, W2, b2, gamma, beta, eps, Wp, bp):
    n, d = seq1.shape
    num_layers = W1.shape[0]
    bm = 400 if n % 400 == 0 else n
    pk = jnp.asarray(_pack_matrix(min(_PACK_G, n)))

    cvec0 = jnp.broadcast_to(1.0 + eps[0], (1, d)).astype(jnp.float32)
    x, stats, pack = _layer1(adj, seq1, cvec0, pk, W1[0], b1[0].reshape(1, d),
                             W2[0], b2[0].reshape(1, d), bm)

    for i in range(1, num_layers):
        h = pl.pallas_call(
            _bn_body,
            out_shape=jax.ShapeDtypeStruct((n, d), jnp.float32),
        )(x, stats, gamma[i - 1].reshape(1, d), beta[i - 1].reshape(1, d))
        hp = h.reshape(n // 16, 16, d).transpose(1, 0, 2)
        hphi = hp.astype(jnp.bfloat16)
        hplo = (hp - hphi.astype(jnp.float32)).astype(jnp.bfloat16)
        cvec = jnp.broadcast_to(1.0 + eps[i], (1, d)).astype(jnp.float32)
        x, stats = _layer2(pack, hphi, hplo, h, cvec, W1[i],
                           b1[i].reshape(1, d), W2[i], b2[i].reshape(1, d), bm)

    out = pl.pallas_call(
        _bn_head_body,
        out_shape=jax.ShapeDtypeStruct((n, 1), jnp.float32),
    )(x, stats, gamma[num_layers - 1].reshape(1, d),
      beta[num_layers - 1].reshape(1, d), Wp, bp.reshape(1, 1))
    return out


# speedup vs baseline: 1.0390x; 1.0390x over previous
import functools

import jax
import jax.numpy as jnp
import numpy as np
from jax.experimental import pallas as pl

_PACK_G = 2048


def _pack_matrix(width: int) -> np.ndarray:
    pk = np.zeros((width, width // 16), np.float32)
    j = np.arange(width)
    pk[j, j // 16] = 2.0 ** (j % 16)
    return pk


def _layer1_body(adj_ref, h_ref, hblk_ref, cvec_ref, pk_ref, w1_ref, b1_ref,
                 w2_ref, b2_ref, x_ref, stats_ref, pack_ref):
    adj = adj_ref[...].astype(jnp.bfloat16)
    n = adj.shape[1]
    h = h_ref[...]
    hhi = h.astype(jnp.bfloat16)
    hlo = (h - hhi.astype(jnp.float32)).astype(jnp.bfloat16)
    pooled = jnp.dot(adj, hhi, preferred_element_type=jnp.float32)
    pooled = pooled + jnp.dot(adj, hlo, preferred_element_type=jnp.float32)
    pooled = pooled + cvec_ref[...] * hblk_ref[...]
    t = jnp.dot(pooled, w1_ref[...], preferred_element_type=jnp.float32)
    t = jnp.maximum(t + b1_ref[...], 0.0)
    x = jnp.dot(t, w2_ref[...], preferred_element_type=jnp.float32)
    x = x + b2_ref[...]
    x_ref[...] = x
    stats_ref[...] = jnp.stack([jnp.sum(x, axis=0),
                                jnp.sum(x * x, axis=0)])[None]
    parts = []
    for g0 in range(0, n, _PACK_G):
        w = min(_PACK_G, n - g0)
        parts.append(jnp.dot(adj[:, g0:g0 + w],
                             pk_ref[:w, :w // 16].astype(jnp.bfloat16),
                             preferred_element_type=jnp.float32))
    pack_ref[...] = jnp.concatenate(parts, axis=1).astype(jnp.int32)


def _layer2_body(pack_ref, hphi_ref, hplo_ref, hblk_ref, cvec_ref, w1_ref,
                 b1_ref, w2_ref, b2_ref, x_ref, stats_ref):
    bits = pack_ref[...]
    bm = bits.shape[0]
    acc = jnp.zeros((bm, hphi_ref.shape[2]), jnp.float32)
    for k in range(16):
        if k:
            bits = jax.lax.shift_right_logical(bits, 1)
        v = (bits & 1).astype(jnp.bfloat16)
        acc = acc + jnp.dot(v, hphi_ref[k], preferred_element_type=jnp.float32)
        acc = acc + jnp.dot(v, hplo_ref[k], preferred_element_type=jnp.float32)
    pooled = acc + cvec_ref[...] * hblk_ref[...]
    t = jnp.dot(pooled, w1_ref[...], preferred_element_type=jnp.float32)
    t = jnp.maximum(t + b1_ref[...], 0.0)
    x = jnp.dot(t, w2_ref[...], preferred_element_type=jnp.float32)
    x = x + b2_ref[...]
    x_ref[...] = x
    stats_ref[...] = jnp.stack([jnp.sum(x, axis=0),
                                jnp.sum(x * x, axis=0)])[None]


def _bn_body(x_ref, stats_ref, gamma_ref, beta_ref, h_ref):
    n = x_ref.shape[0]
    s = jnp.sum(stats_ref[...], axis=0)
    m = s[0:1] * (1.0 / n)
    var = s[1:2] * (1.0 / n) - m * m
    inv = gamma_ref[...] * jax.lax.rsqrt(var + 1e-5)
    h_ref[...] = jnp.maximum((x_ref[...] - m) * inv + beta_ref[...], 0.0)


def _bn_head_body(x_ref, stats_ref, gamma_ref, beta_ref, wp_ref, bp_ref,
                  out_ref):
    n = x_ref.shape[0]
    s = jnp.sum(stats_ref[...], axis=0)
    m = s[0:1] * (1.0 / n)
    var = s[1:2] * (1.0 / n) - m * m
    inv = gamma_ref[...] * jax.lax.rsqrt(var + 1e-5)
    h = jnp.maximum((x_ref[...] - m) * inv + beta_ref[...], 0.0)
    out_ref[...] = jnp.dot(h, wp_ref[...],
                           preferred_element_type=jnp.float32) + bp_ref[...]


def _mlp_specs(d):
    return [
        pl.BlockSpec((1, d), lambda i: (0, 0)),
        pl.BlockSpec((d, d), lambda i: (0, 0)),
        pl.BlockSpec((1, d), lambda i: (0, 0)),
        pl.BlockSpec((d, d), lambda i: (0, 0)),
        pl.BlockSpec((1, d), lambda i: (0, 0)),
    ]


def _layer1(adj, h, cvec, pk, w1, b1, w2, b2, bm):
    n, d = h.shape
    nb = n // bm
    np16 = n // 16
    pkw = pk.shape[0]
    return pl.pallas_call(
        _layer1_body,
        grid=(nb,),
        in_specs=[
            pl.BlockSpec((bm, n), lambda i: (i, 0)),
            pl.BlockSpec((n, d), lambda i: (0, 0)),
            pl.BlockSpec((bm, d), lambda i: (i, 0)),
            pl.BlockSpec((1, d), lambda i: (0, 0)),
            pl.BlockSpec((pkw, pkw // 16), lambda i: (0, 0)),
            pl.BlockSpec((d, d), lambda i: (0, 0)),
            pl.BlockSpec((1, d), lambda i: (0, 0)),
            pl.BlockSpec((d, d), lambda i: (0, 0)),
            pl.BlockSpec((1, d), lambda i: (0, 0)),
        ],
        out_specs=[
            pl.BlockSpec((bm, d), lambda i: (i, 0)),
            pl.BlockSpec((1, 2, d), lambda i: (i, 0, 0)),
            pl.BlockSpec((bm, np16), lambda i: (i, 0)),
        ],
        out_shape=[
            jax.ShapeDtypeStruct((n, d), jnp.float32),
            jax.ShapeDtypeStruct((nb, 2, d), jnp.float32),
            jax.ShapeDtypeStruct((n, np16), jnp.int32),
        ],
    )(adj, h, h, cvec, pk, w1, b1, w2, b2)


def _layer2(pack, hphi, hplo, h, cvec, w1, b1, w2, b2, bm):
    n, d = h.shape
    nb = n // bm
    np16 = n // 16
    return pl.pallas_call(
        _layer2_body,
        grid=(nb,),
        in_specs=[
            pl.BlockSpec((bm, np16), lambda i: (i, 0)),
            pl.BlockSpec((16, np16, d), lambda i: (0, 0, 0)),
            pl.BlockSpec((16, np16, d), lambda i: (0, 0, 0)),
            pl.BlockSpec((bm, d), lambda i: (i, 0)),
        ] + _mlp_specs(d),
        out_specs=[
            pl.BlockSpec((bm, d), lambda i: (i, 0)),
            pl.BlockSpec((1, 2, d), lambda i: (i, 0, 0)),
        ],
        out_shape=[
            jax.ShapeDtypeStruct((n, d), jnp.float32),
            jax.ShapeDtypeStruct((nb, 2, d), jnp.float32),
        ],
    )(pack, hphi, hplo, h, cvec, w1, b1, w2, b2)


def kernel(seq1, adj, W1, b1, W2, b2, gamma, beta, eps, Wp, bp):
    n, d = seq1.shape
    num_layers = W1.shape[0]
    bm = 400 if n % 400 == 0 else n
    pk = jnp.asarray(_pack_matrix(min(_PACK_G, n)))

    cvec0 = jnp.broadcast_to(1.0 + eps[0], (1, d)).astype(jnp.float32)
    x, stats, pack = _layer1(adj, seq1, cvec0, pk, W1[0], b1[0].reshape(1, d),
                             W2[0], b2[0].reshape(1, d), bm)

    for i in range(1, num_layers):
        h = pl.pallas_call(
            _bn_body,
            out_shape=jax.ShapeDtypeStruct((n, d), jnp.float32),
        )(x, stats, gamma[i - 1].reshape(1, d), beta[i - 1].reshape(1, d))
        hp = h.reshape(n // 16, 16, d).transpose(1, 0, 2)
        hphi = hp.astype(jnp.bfloat16)
        hplo = (hp - hphi.astype(jnp.float32)).astype(jnp.bfloat16)
        cvec = jnp.broadcast_to(1.0 + eps[i], (1, d)).astype(jnp.float32)
        x, stats = _layer2(pack, hphi, hplo, h, cvec, W1[i],
                           b1[i].reshape(1, d), W2[i], b2[i].reshape(1, d), bm)

    out = pl.pallas_call(
        _bn_head_body,
        out_shape=jax.ShapeDtypeStruct((n, 1), jnp.float32),
    )(x, stats, gamma[num_layers - 1].reshape(1, d),
      beta[num_layers - 1].reshape(1, d), Wp, bp.reshape(1, 1))
    return out
